# Initial kernel scaffold; baseline (speedup 1.0000x reference)
import jax
import jax.numpy as jnp
from jax.experimental import pallas as pl


def kernel(x, edge_attr, edge_direction, simplified_edge_batch, simplified_edge_index, batch, params):
    raise NotImplementedError("write your pallas kernel here")


# speedup vs baseline: 1.0001x; 1.0001x over previous
import functools

import jax
import jax.numpy as jnp
from jax.experimental import pallas as pl
from jax.experimental.pallas import tpu as pltpu

N = 10000
E = 320000
M = 160000
G = 64
NHID = 256
NOUT = 128
NLAYER = 3


def _readout_body(g_ref, wo1_ref, bo1_ref, lng_ref, lnb_ref, wo2_ref, bo2_ref, o_ref):
    g = g_ref[...]
    o = g @ wo1_ref[...] + bo1_ref[...]
    mu = jnp.mean(o, axis=-1, keepdims=True)
    var = jnp.mean((o - mu) ** 2, axis=-1, keepdims=True)
    o = (o - mu) * jax.lax.rsqrt(var + 1e-5) * lng_ref[...] + lnb_ref[...]
    o = jnp.maximum(o, 0.0)
    o_ref[...] = o @ wo2_ref[...] + bo2_ref[...]


def _readout(g, params):
    return pl.pallas_call(
        _readout_body,
        out_shape=jax.ShapeDtypeStruct((G, NOUT), jnp.float32),
    )(g, params['Wo1'], params['bo1'][None, :], params['ln_g'][None, :],
      params['ln_b'][None, :], params['Wo2'], params['bo2'][None, :])


def kernel(x, edge_attr, edge_direction, simplified_edge_batch, simplified_edge_index, batch, params):
    h = x @ params['W_node'] + params['b_node']
    e = jax.nn.relu(edge_attr @ params['We1'] + params['be1'])
    e = jax.nn.relu(e @ params['We2'] + params['be2'])
    r = e @ params['We3'] + params['be3'] + jnp.take(params['emb_dir'], edge_direction, axis=0)
    s2 = jax.ops.segment_sum(r, simplified_edge_batch, num_segments=M)
    se = s2 @ params['Wet'] + params['bet']
    src = simplified_edge_index[0]
    dst = simplified_edge_index[1]
    for l in range(NLAYER):
        msg = jax.nn.relu(jnp.take(h, src, axis=0) + se)
        agg = jax.ops.segment_sum(msg, dst, num_segments=N)
        h = jax.nn.relu(agg @ params['Wc'][l] + params['bc'][l]) + h
    g = jax.ops.segment_sum(h, batch, num_segments=G)
    return _readout(g, params)


# speedup vs baseline: 1.9730x; 1.9729x over previous
import functools

import jax
import jax.numpy as jnp
from jax import lax
from jax.experimental import pallas as pl
from jax.experimental.pallas import tpu as pltpu
from jax.experimental.pallas import tpu_sc as plsc

N = 10000
E = 320000
M = 160000
G = 64
NHID = 256
NOUT = 128
NLAYER = 3

NC = 2
NS = 16
HALF = 128

SEGC = 12288
NPASS = (M + SEGC - 1) // SEGC
M_PAD = NPASS * SEGC
EG = E // 128
MG = M // 128

TE = 512
ACC_B = SEGC + 64
ACC_D = 10240

_f32 = jnp.float32
_i32 = jnp.int32


def _node_enc_body(x_ref, w_ref, b_ref, h_ref):
    h = jnp.dot(x_ref[...], w_ref[...], preferred_element_type=_f32) + b_ref[...]
    h_ref[0] = h[:, :HALF]
    h_ref[1] = h[:, HALF:]


def _node_enc(x, w, b):
    return pl.pallas_call(
        _node_enc_body,
        out_shape=jax.ShapeDtypeStruct((NC, N, HALF), _f32),
    )(x, w, b[None, :])


def _edge_mlp_body(ea_ref, dir_ref, seb_ref, w1_ref, b1_ref, w2_ref, b2_ref,
                   w3_ref, b3_ref, emb_ref, r_ref, bounds_ref):
    i = pl.program_id(0)
    a = ea_ref[...]
    e1 = jnp.maximum(jnp.dot(a, w1_ref[...], preferred_element_type=_f32) + b1_ref[...], 0.0)
    e2 = jnp.maximum(jnp.dot(e1, w2_ref[...], preferred_element_type=_f32) + b2_ref[...], 0.0)
    d = dir_ref[...]
    oh = (d == lax.broadcasted_iota(_i32, (TE, 8), 1)).astype(_f32)
    r = (jnp.dot(e2, w3_ref[...], preferred_element_type=_f32) + b3_ref[...]
         + jnp.dot(oh, emb_ref[...], preferred_element_type=_f32))
    r_ref[0] = r[:, :HALF]
    r_ref[1] = r[:, HALF:]
    sb = seb_ref[...]
    thr = lax.broadcasted_iota(_i32, (1, 128), 1) * SEGC
    cnt = jnp.sum((sb < thr).astype(_i32), axis=0, keepdims=True)

    @pl.when(i == 0)
    def _():
        bounds_ref[...] = jnp.zeros_like(bounds_ref)

    bounds_ref[...] += cnt


def _edge_mlp(edge_attr, edge_direction, seb, params):
    nb = E // TE
    emb8 = jnp.zeros((8, NHID), _f32).at[:4].set(params['emb_dir'])
    full = lambda shape: pl.BlockSpec(shape, lambda i: (0,) * len(shape))
    return pl.pallas_call(
        _edge_mlp_body,
        grid=(nb,),
        in_specs=[
            pl.BlockSpec((TE, 16), lambda i: (i, 0)),
            pl.BlockSpec((TE, 1), lambda i: (i, 0)),
            pl.BlockSpec((TE, 1), lambda i: (i, 0)),
            full((16, NHID)), full((1, NHID)),
            full((NHID, NHID)), full((1, NHID)),
            full((NHID, NHID)), full((1, NHID)),
            full((8, NHID)),
        ],
        out_specs=[
            pl.BlockSpec((NC, TE, HALF), lambda i: (0, i, 0)),
            pl.BlockSpec((1, 128), lambda i: (0, 0)),
        ],
        out_shape=[
            jax.ShapeDtypeStruct((NC, E, HALF), _f32),
            jax.ShapeDtypeStruct((1, 128), _i32),
        ],
    )(edge_attr, edge_direction[:, None].astype(_i32), seb[:, None],
      params['We1'], params['be1'][None, :], params['We2'], params['be2'][None, :],
      params['We3'], params['be3'][None, :], emb8)


def _edge_tf_body(s2_ref, w_ref, b_ref, se_ref):
    lo = jnp.dot(s2_ref[0], w_ref[:HALF, :], preferred_element_type=_f32)
    hi = jnp.dot(s2_ref[1], w_ref[HALF:, :], preferred_element_type=_f32)
    se = lo + hi + b_ref[...]
    se_ref[0] = se[:, :HALF]
    se_ref[1] = se[:, HALF:]


def _edge_tf(s2_pad, w, b):
    tm = 640
    return pl.pallas_call(
        _edge_tf_body,
        grid=(M // tm,),
        in_specs=[
            pl.BlockSpec((NC, tm, HALF), lambda i: (0, i, 0)),
            pl.BlockSpec((NHID, NHID), lambda i: (0, 0)),
            pl.BlockSpec((1, NHID), lambda i: (0, 0)),
        ],
        out_specs=pl.BlockSpec((NC, tm, HALF), lambda i: (0, i, 0)),
        out_shape=jax.ShapeDtypeStruct((NC, M, HALF), _f32),
    )(s2_pad, w, b[None, :])


def _conv_dense_body(agg_ref, h_ref, w_ref, b_ref, out_ref):
    lo = jnp.dot(agg_ref[0], w_ref[:HALF, :], preferred_element_type=_f32)
    hi = jnp.dot(agg_ref[1], w_ref[HALF:, :], preferred_element_type=_f32)
    hn = jnp.maximum(lo + hi + b_ref[...], 0.0)
    out_ref[0] = hn[:, :HALF] + h_ref[0]
    out_ref[1] = hn[:, HALF:] + h_ref[1]


def _conv_dense(agg3, h3, w, b):
    tn = 1000
    return pl.pallas_call(
        _conv_dense_body,
        grid=(N // tn,),
        in_specs=[
            pl.BlockSpec((NC, tn, HALF), lambda i: (0, i, 0)),
            pl.BlockSpec((NC, tn, HALF), lambda i: (0, i, 0)),
            pl.BlockSpec((NHID, NHID), lambda i: (0, 0)),
            pl.BlockSpec((1, NHID), lambda i: (0, 0)),
        ],
        out_specs=pl.BlockSpec((NC, tn, HALF), lambda i: (0, i, 0)),
        out_shape=jax.ShapeDtypeStruct((NC, N, HALF), _f32),
    )(agg3, h3, w, b[None, :])


def _readout_body(h_ref, b_ref, wo1_ref, bo1_ref, lng_ref, lnb_ref, wo2_ref,
                  bo2_ref, o_ref):
    bo = (b_ref[...] == lax.broadcasted_iota(_i32, (N, G), 1)).astype(_f32)
    dn = (((0,), (0,)), ((), ()))
    g_lo = lax.dot_general(bo, h_ref[0], dn, preferred_element_type=_f32)
    g_hi = lax.dot_general(bo, h_ref[1], dn, preferred_element_type=_f32)
    o = (jnp.dot(g_lo, wo1_ref[:HALF, :], preferred_element_type=_f32)
         + jnp.dot(g_hi, wo1_ref[HALF:, :], preferred_element_type=_f32)
         + bo1_ref[...])
    mu = jnp.mean(o, axis=-1, keepdims=True)
    var = jnp.mean((o - mu) ** 2, axis=-1, keepdims=True)
    o = (o - mu) * lax.rsqrt(var + 1e-5) * lng_ref[...] + lnb_ref[...]
    o = jnp.maximum(o, 0.0)
    o_ref[...] = jnp.dot(o, wo2_ref[...], preferred_element_type=_f32) + bo2_ref[...]


def _readout(h3, batch, params):
    return pl.pallas_call(
        _readout_body,
        out_shape=jax.ShapeDtypeStruct((G, NOUT), _f32),
    )(h3, batch[:, None], params['Wo1'], params['bo1'][None, :],
      params['ln_g'][None, :], params['ln_b'][None, :],
      params['Wo2'], params['bo2'][None, :])


def _zero_zbuf(zbuf):
    def zrow(i, _):
        for u in range(8):
            zbuf[i, pl.ds(u * 16, 16)] = jnp.zeros((16,), _f32)
        return 0
    lax.fori_loop(0, zbuf.shape[0], zrow, 0, unroll=False)


def _sc_mesh():
    return plsc.VectorSubcoreMesh(core_axis_name="c", subcore_axis_name="s")


def _segsum_e2m_body(r_ref, seb_ref, bounds_ref, s2_ref,
                     acc, zbuf, bnd_v, seb_v, idx_v, rows_v):
    c = lax.axis_index("c")
    s = lax.axis_index("s")
    pltpu.sync_copy(bounds_ref, bnd_v)
    _zero_zbuf(zbuf)

    for p in range(NPASS):
        def zc(j, _):
            @pl.when(j % NS == s)
            def _():
                pltpu.sync_copy(zbuf, acc.at[pl.ds(j * 64, 64)])
            return 0
        lax.fori_loop(0, ACC_B // 64, zc, 0, unroll=False)
        plsc.subcore_barrier()

        lo = bnd_v[0, pl.ds((p // 16) * 16, 16)][p % 16]
        hi = bnd_v[0, pl.ds(((p + 1) // 16) * 16, 16)][(p + 1) % 16]
        g_lo = lo // 128
        g_hi = (hi + 127) // 128
        base = p * SEGC
        nt = jnp.maximum(g_hi - g_lo - s + NS - 1, 0) // NS

        def chunk(t, _):
            g = g_lo + s + t * NS
            pltpu.sync_copy(seb_ref.at[pl.ds(g * 128, 128)], seb_v)
            for u in range(8):
                v = seb_v[pl.ds(u * 16, 16)] - base
                ok = (v >= 0) & (v < SEGC)
                idx_v[pl.ds(u * 16, 16)] = jnp.where(ok, v, SEGC)
            pltpu.sync_copy(r_ref.at[pl.ds(c * E + g * 128, 128)], rows_v)
            pltpu.sync_copy(rows_v, acc.at[idx_v], add=True)
            return 0
        lax.fori_loop(0, nt, chunk, 0, unroll=False)
        plsc.subcore_barrier()
        r0 = s * (SEGC // NS)
        o0 = c * M_PAD + base + r0
        pltpu.sync_copy(acc.at[pl.ds(r0, 512)], s2_ref.at[pl.ds(o0, 512)])
        pltpu.sync_copy(acc.at[pl.ds(r0 + 512, 256)],
                        s2_ref.at[pl.ds(o0 + 512, 256)])
        plsc.subcore_barrier()


def _segsum_e2m(r_flat, seb2, bounds):
    k = pl.kernel(
        _segsum_e2m_body,
        out_type=jax.ShapeDtypeStruct((NC * M_PAD, HALF), _f32),
        mesh=_sc_mesh(),
        scratch_types=[
            pltpu.VMEM_SHARED((ACC_B, HALF), _f32),
            pltpu.VMEM((64, HALF), _f32),
            pltpu.VMEM((1, 128), _i32),
            pltpu.VMEM((128,), _i32),
            pltpu.VMEM((128,), _i32),
            pltpu.VMEM((128, HALF), _f32),
        ],
    )
    return k(r_flat, seb2, bounds)


def _conv_gather_body(h_ref, se_ref, src_ref, dst_ref, agg_ref,
                      acc, zbuf, sidx_v, didx_v, rows_v, se_v, sem):
    c = lax.axis_index("c")
    s = lax.axis_index("s")
    _zero_zbuf(zbuf)

    def zc(j, _):
        @pl.when(j % NS == s)
        def _():
            pltpu.sync_copy(zbuf, acc.at[pl.ds(j * 64, 64)])
        return 0
    lax.fori_loop(0, ACC_D // 64, zc, 0, unroll=False)
    plsc.subcore_barrier()

    nt = (MG + NS - 1) // NS

    def chunk(t, _):
        g = s + t * NS

        @pl.when(g < MG)
        def _():
            pltpu.sync_copy(src_ref.at[pl.ds(g * 128, 128)], sidx_v)
            pltpu.sync_copy(dst_ref.at[pl.ds(g * 128, 128)], didx_v)
            off = c * N
            for u in range(8):
                sidx_v[pl.ds(u * 16, 16)] = sidx_v[pl.ds(u * 16, 16)] + off
            pltpu.async_copy(h_ref.at[sidx_v], rows_v, sem).wait()
            pltpu.sync_copy(se_ref.at[pl.ds(c * M + g * 128, 128)], se_v)

            def fuse(i, _):
                for u in range(8):
                    sl = pl.ds(u * 16, 16)
                    rows_v[i, sl] = jnp.maximum(rows_v[i, sl] + se_v[i, sl], 0.0)
                return 0
            lax.fori_loop(0, 128, fuse, 0, unroll=False)
            pltpu.sync_copy(rows_v, acc.at[didx_v], add=True)
        return 0

    lax.fori_loop(0, nt, chunk, 0, unroll=False)
    plsc.subcore_barrier()
    r0 = s * 640

    @pl.when(s < NS - 1)
    def _():
        pltpu.sync_copy(acc.at[pl.ds(r0, 640)],
                        agg_ref.at[pl.ds(c * N + r0, 640)])

    @pl.when(s == NS - 1)
    def _():
        pltpu.sync_copy(acc.at[pl.ds(r0, 400)],
                        agg_ref.at[pl.ds(c * N + r0, 400)])


def _conv_gather(h_flat, se_flat, src2, dst2):
    k = pl.kernel(
        _conv_gather_body,
        out_type=jax.ShapeDtypeStruct((NC * N, HALF), _f32),
        mesh=_sc_mesh(),
        scratch_types=[
            pltpu.VMEM_SHARED((ACC_D, HALF), _f32),
            pltpu.VMEM((64, HALF), _f32),
            pltpu.VMEM((128,), _i32),
            pltpu.VMEM((128,), _i32),
            pltpu.VMEM((128, HALF), _f32),
            pltpu.VMEM((128, HALF), _f32),
            pltpu.SemaphoreType.DMA,
        ],
    )
    return k(h_flat, se_flat, src2, dst2)


def kernel(x, edge_attr, edge_direction, simplified_edge_batch, simplified_edge_index, batch, params):
    seb = simplified_edge_batch.astype(_i32)
    h3 = _node_enc(x, params['W_node'], params['b_node'])
    r3, bounds = _edge_mlp(edge_attr, edge_direction, seb, params)
    s2_flat = _segsum_e2m(r3.reshape(NC * E, HALF), seb, bounds)
    se3 = _edge_tf(s2_flat.reshape(NC, M_PAD, HALF), params['Wet'], params['bet'])
    src2 = simplified_edge_index[0].astype(_i32)
    dst2 = simplified_edge_index[1].astype(_i32)
    se_flat = se3.reshape(NC * M, HALF)
    for l in range(NLAYER):
        agg = _conv_gather(h3.reshape(NC * N, HALF), se_flat, src2, dst2)
        h3 = _conv_dense(agg.reshape(NC, N, HALF), h3, params['Wc'][l], params['bc'][l])
    return _readout(h3, batch.astype(_i32), params)


# speedup vs baseline: 1.9762x; 1.0016x over previous
import functools

import jax
import jax.numpy as jnp
from jax import lax
from jax.experimental import pallas as pl
from jax.experimental.pallas import tpu as pltpu
from jax.experimental.pallas import tpu_sc as plsc

N = 10000
E = 320000
M = 160000
G = 64
NHID = 256
NOUT = 128
NLAYER = 3

NC = 2
NS = 16
HALF = 128

SEGC = 12288
NPASS = (M + SEGC - 1) // SEGC
M_PAD = NPASS * SEGC
EG = E // 128
MG = M // 128

TE = 512
ACC_B = SEGC + 64
ACC_D = 10240

_f32 = jnp.float32
_bf16 = jnp.bfloat16
_i32 = jnp.int32


def _node_enc_body(x_ref, w_ref, b_ref, h_ref):
    h = jnp.dot(x_ref[...], w_ref[...], preferred_element_type=_f32) + b_ref[...]
    h_ref[0] = h[:, :HALF]
    h_ref[1] = h[:, HALF:]


def _node_enc(x, w, b):
    return pl.pallas_call(
        _node_enc_body,
        out_shape=jax.ShapeDtypeStruct((NC, N, HALF), _f32),
    )(x, w, b[None, :])


def _edge_mlp_body(ea_ref, dir_ref, seb_ref, w1_ref, b1_ref, w2_ref, b2_ref,
                   w3_ref, b3_ref, emb_ref, r_ref, bounds_ref):
    i = pl.program_id(0)
    a = ea_ref[...]
    e1 = jnp.maximum(jnp.dot(a, w1_ref[...], preferred_element_type=_f32) + b1_ref[...], 0.0)
    e2 = jnp.maximum(jnp.dot(e1.astype(_bf16), w2_ref[...], preferred_element_type=_f32) + b2_ref[...], 0.0)
    d = dir_ref[...]
    oh = (d == lax.broadcasted_iota(_i32, (TE, 8), 1)).astype(_f32)
    r = (jnp.dot(e2.astype(_bf16), w3_ref[...], preferred_element_type=_f32) + b3_ref[...]
         + jnp.dot(oh, emb_ref[...], preferred_element_type=_f32))
    r_ref[0] = r[:, :HALF]
    r_ref[1] = r[:, HALF:]
    sb = seb_ref[...]
    thr = lax.broadcasted_iota(_i32, (1, 128), 1) * SEGC
    cnt = jnp.sum((sb < thr).astype(_i32), axis=0, keepdims=True)

    @pl.when(i == 0)
    def _():
        bounds_ref[...] = jnp.zeros_like(bounds_ref)

    bounds_ref[...] += cnt


def _edge_mlp(edge_attr, edge_direction, seb, params):
    nb = E // TE
    emb8 = jnp.zeros((8, NHID), _f32).at[:4].set(params['emb_dir'])
    full = lambda shape: pl.BlockSpec(shape, lambda i: (0,) * len(shape))
    return pl.pallas_call(
        _edge_mlp_body,
        grid=(nb,),
        in_specs=[
            pl.BlockSpec((TE, 16), lambda i: (i, 0)),
            pl.BlockSpec((TE, 1), lambda i: (i, 0)),
            pl.BlockSpec((TE, 1), lambda i: (i, 0)),
            full((16, NHID)), full((1, NHID)),
            full((NHID, NHID)), full((1, NHID)),
            full((NHID, NHID)), full((1, NHID)),
            full((8, NHID)),
        ],
        out_specs=[
            pl.BlockSpec((NC, TE, HALF), lambda i: (0, i, 0)),
            pl.BlockSpec((1, 128), lambda i: (0, 0)),
        ],
        out_shape=[
            jax.ShapeDtypeStruct((NC, E, HALF), _f32),
            jax.ShapeDtypeStruct((1, 128), _i32),
        ],
    )(edge_attr, edge_direction[:, None].astype(_i32), seb[:, None],
      params['We1'], params['be1'][None, :],
      params['We2'].astype(_bf16), params['be2'][None, :],
      params['We3'].astype(_bf16), params['be3'][None, :], emb8)


def _edge_tf_body(s2_ref, w_ref, b_ref, se_ref):
    lo = jnp.dot(s2_ref[0], w_ref[:HALF, :], preferred_element_type=_f32)
    hi = jnp.dot(s2_ref[1], w_ref[HALF:, :], preferred_element_type=_f32)
    se = lo + hi + b_ref[...]
    se_ref[0] = se[:, :HALF]
    se_ref[1] = se[:, HALF:]


def _edge_tf(s2_pad, w, b):
    tm = 640
    return pl.pallas_call(
        _edge_tf_body,
        grid=(M // tm,),
        in_specs=[
            pl.BlockSpec((NC, tm, HALF), lambda i: (0, i, 0)),
            pl.BlockSpec((NHID, NHID), lambda i: (0, 0)),
            pl.BlockSpec((1, NHID), lambda i: (0, 0)),
        ],
        out_specs=pl.BlockSpec((NC, tm, HALF), lambda i: (0, i, 0)),
        out_shape=jax.ShapeDtypeStruct((NC, M, HALF), _f32),
    )(s2_pad, w, b[None, :])


def _conv_dense_body(agg_ref, h_ref, w_ref, b_ref, out_ref):
    lo = jnp.dot(agg_ref[0], w_ref[:HALF, :], preferred_element_type=_f32)
    hi = jnp.dot(agg_ref[1], w_ref[HALF:, :], preferred_element_type=_f32)
    hn = jnp.maximum(lo + hi + b_ref[...], 0.0)
    out_ref[0] = hn[:, :HALF] + h_ref[0]
    out_ref[1] = hn[:, HALF:] + h_ref[1]


def _conv_dense(agg3, h3, w, b):
    tn = 1000
    return pl.pallas_call(
        _conv_dense_body,
        grid=(N // tn,),
        in_specs=[
            pl.BlockSpec((NC, tn, HALF), lambda i: (0, i, 0)),
            pl.BlockSpec((NC, tn, HALF), lambda i: (0, i, 0)),
            pl.BlockSpec((NHID, NHID), lambda i: (0, 0)),
            pl.BlockSpec((1, NHID), lambda i: (0, 0)),
        ],
        out_specs=pl.BlockSpec((NC, tn, HALF), lambda i: (0, i, 0)),
        out_shape=jax.ShapeDtypeStruct((NC, N, HALF), _f32),
    )(agg3, h3, w, b[None, :])


def _readout_body(h_ref, b_ref, wo1_ref, bo1_ref, lng_ref, lnb_ref, wo2_ref,
                  bo2_ref, o_ref):
    bo = (b_ref[...] == lax.broadcasted_iota(_i32, (N, G), 1)).astype(_f32)
    dn = (((0,), (0,)), ((), ()))
    g_lo = lax.dot_general(bo, h_ref[0], dn, preferred_element_type=_f32)
    g_hi = lax.dot_general(bo, h_ref[1], dn, preferred_element_type=_f32)
    o = (jnp.dot(g_lo, wo1_ref[:HALF, :], preferred_element_type=_f32)
         + jnp.dot(g_hi, wo1_ref[HALF:, :], preferred_element_type=_f32)
         + bo1_ref[...])
    mu = jnp.mean(o, axis=-1, keepdims=True)
    var = jnp.mean((o - mu) ** 2, axis=-1, keepdims=True)
    o = (o - mu) * lax.rsqrt(var + 1e-5) * lng_ref[...] + lnb_ref[...]
    o = jnp.maximum(o, 0.0)
    o_ref[...] = jnp.dot(o, wo2_ref[...], preferred_element_type=_f32) + bo2_ref[...]


def _readout(h3, batch, params):
    return pl.pallas_call(
        _readout_body,
        out_shape=jax.ShapeDtypeStruct((G, NOUT), _f32),
    )(h3, batch[:, None], params['Wo1'], params['bo1'][None, :],
      params['ln_g'][None, :], params['ln_b'][None, :],
      params['Wo2'], params['bo2'][None, :])


def _zero_zbuf(zbuf):
    def zrow(i, _):
        for u in range(8):
            zbuf[i, pl.ds(u * 16, 16)] = jnp.zeros((16,), _f32)
        return 0
    lax.fori_loop(0, zbuf.shape[0], zrow, 0, unroll=False)


def _sc_mesh():
    return plsc.VectorSubcoreMesh(core_axis_name="c", subcore_axis_name="s")


def _segsum_e2m_body(r_ref, seb_ref, bounds_ref, s2_ref,
                     acc, zbuf, bnd_v, seb_v, idx_v, rows_v):
    c = lax.axis_index("c")
    s = lax.axis_index("s")
    pltpu.sync_copy(bounds_ref, bnd_v)
    _zero_zbuf(zbuf)

    for p in range(NPASS):
        def zc(j, _):
            @pl.when(j % NS == s)
            def _():
                pltpu.sync_copy(zbuf, acc.at[pl.ds(j * 64, 64)])
            return 0
        lax.fori_loop(0, ACC_B // 64, zc, 0, unroll=False)
        plsc.subcore_barrier()

        lo = bnd_v[0, pl.ds((p // 16) * 16, 16)][p % 16]
        hi = bnd_v[0, pl.ds(((p + 1) // 16) * 16, 16)][(p + 1) % 16]
        g_lo = lo // 128
        g_hi = (hi + 127) // 128
        base = p * SEGC
        nt = jnp.maximum(g_hi - g_lo - s + NS - 1, 0) // NS

        def chunk(t, _):
            g = g_lo + s + t * NS
            pltpu.sync_copy(seb_ref.at[pl.ds(g * 128, 128)], seb_v)
            for u in range(8):
                v = seb_v[pl.ds(u * 16, 16)] - base
                ok = (v >= 0) & (v < SEGC)
                idx_v[pl.ds(u * 16, 16)] = jnp.where(ok, v, SEGC)
            pltpu.sync_copy(r_ref.at[pl.ds(c * E + g * 128, 128)], rows_v)
            pltpu.sync_copy(rows_v, acc.at[idx_v], add=True)
            return 0
        lax.fori_loop(0, nt, chunk, 0, unroll=False)
        plsc.subcore_barrier()
        r0 = s * (SEGC // NS)
        o0 = c * M_PAD + base + r0
        pltpu.sync_copy(acc.at[pl.ds(r0, 512)], s2_ref.at[pl.ds(o0, 512)])
        pltpu.sync_copy(acc.at[pl.ds(r0 + 512, 256)],
                        s2_ref.at[pl.ds(o0 + 512, 256)])
        plsc.subcore_barrier()


def _segsum_e2m(r_flat, seb2, bounds):
    k = pl.kernel(
        _segsum_e2m_body,
        out_type=jax.ShapeDtypeStruct((NC * M_PAD, HALF), _f32),
        mesh=_sc_mesh(),
        scratch_types=[
            pltpu.VMEM_SHARED((ACC_B, HALF), _f32),
            pltpu.VMEM((64, HALF), _f32),
            pltpu.VMEM((1, 128), _i32),
            pltpu.VMEM((128,), _i32),
            pltpu.VMEM((128,), _i32),
            pltpu.VMEM((128, HALF), _f32),
        ],
    )
    return k(r_flat, seb2, bounds)


def _conv_gather_body(h_ref, se_ref, src_ref, dst_ref, agg_ref,
                      acc, zbuf, sidx_v, didx_v, rows_v, se_v, sem):
    c = lax.axis_index("c")
    s = lax.axis_index("s")
    _zero_zbuf(zbuf)

    def zc(j, _):
        @pl.when(j % NS == s)
        def _():
            pltpu.sync_copy(zbuf, acc.at[pl.ds(j * 64, 64)])
        return 0
    lax.fori_loop(0, ACC_D // 64, zc, 0, unroll=False)
    plsc.subcore_barrier()

    nt = (MG + NS - 1) // NS

    def chunk(t, _):
        g = s + t * NS

        @pl.when(g < MG)
        def _():
            pltpu.sync_copy(src_ref.at[pl.ds(g * 128, 128)], sidx_v)
            pltpu.sync_copy(dst_ref.at[pl.ds(g * 128, 128)], didx_v)
            off = c * N
            for u in range(8):
                sidx_v[pl.ds(u * 16, 16)] = sidx_v[pl.ds(u * 16, 16)] + off
            pltpu.async_copy(h_ref.at[sidx_v], rows_v, sem).wait()
            pltpu.sync_copy(se_ref.at[pl.ds(c * M + g * 128, 128)], se_v)

            def fuse(i, _):
                for u in range(8):
                    sl = pl.ds(u * 16, 16)
                    rows_v[i, sl] = jnp.maximum(rows_v[i, sl] + se_v[i, sl], 0.0)
                return 0
            lax.fori_loop(0, 128, fuse, 0, unroll=False)
            pltpu.sync_copy(rows_v, acc.at[didx_v], add=True)
        return 0

    lax.fori_loop(0, nt, chunk, 0, unroll=False)
    plsc.subcore_barrier()
    r0 = s * 640

    @pl.when(s < NS - 1)
    def _():
        pltpu.sync_copy(acc.at[pl.ds(r0, 640)],
                        agg_ref.at[pl.ds(c * N + r0, 640)])

    @pl.when(s == NS - 1)
    def _():
        pltpu.sync_copy(acc.at[pl.ds(r0, 400)],
                        agg_ref.at[pl.ds(c * N + r0, 400)])


def _conv_gather(h_flat, se_flat, src2, dst2):
    k = pl.kernel(
        _conv_gather_body,
        out_type=jax.ShapeDtypeStruct((NC * N, HALF), _f32),
        mesh=_sc_mesh(),
        scratch_types=[
            pltpu.VMEM_SHARED((ACC_D, HALF), _f32),
            pltpu.VMEM((64, HALF), _f32),
            pltpu.VMEM((128,), _i32),
            pltpu.VMEM((128,), _i32),
            pltpu.VMEM((128, HALF), _f32),
            pltpu.VMEM((128, HALF), _f32),
            pltpu.SemaphoreType.DMA,
        ],
    )
    return k(h_flat, se_flat, src2, dst2)


def kernel(x, edge_attr, edge_direction, simplified_edge_batch, simplified_edge_index, batch, params):
    seb = simplified_edge_batch.astype(_i32)
    h3 = _node_enc(x, params['W_node'], params['b_node'])
    r3, bounds = _edge_mlp(edge_attr, edge_direction, seb, params)
    s2_flat = _segsum_e2m(r3.reshape(NC * E, HALF), seb, bounds)
    se3 = _edge_tf(s2_flat.reshape(NC, M_PAD, HALF), params['Wet'], params['bet'])
    src2 = simplified_edge_index[0].astype(_i32)
    dst2 = simplified_edge_index[1].astype(_i32)
    se_flat = se3.reshape(NC * M, HALF)
    for l in range(NLAYER):
        agg = _conv_gather(h3.reshape(NC * N, HALF), se_flat, src2, dst2)
        h3 = _conv_dense(agg.reshape(NC, N, HALF), h3, params['Wc'][l], params['bc'][l])
    return _readout(h3, batch.astype(_i32), params)
